# baseline (device time: 46256 ns/iter reference)
import jax
import jax.numpy as jnp
from jax import lax
from jax.experimental import pallas as pl
from jax.experimental.pallas import tpu as pltpu

N_DEV = 4
N_COL_BLOCKS = 4


def kernel(x, w_mat, scale_x, scale_w):
    m_total, k_shard = x.shape
    k_total, n = w_mat.shape
    m_per = m_total // N_DEV
    n_blk = n // N_COL_BLOCKS

    def body(x_ref, w_ref, sx_ref, sw_ref, out_ref,
             comm_ref, wv_ref, xl_ref, acc_ref, epi_ref,
             send_sems, recv_sems, wsems, xsem, osems):
        my = lax.axis_index("i")

        barrier_sem = pltpu.get_barrier_semaphore()
        for o in range(1, N_DEV):
            peer = lax.rem(my + o, N_DEV)
            pl.semaphore_signal(
                barrier_sem, inc=1,
                device_id=(peer,), device_id_type=pl.DeviceIdType.MESH,
            )
        pl.semaphore_wait(barrier_sem, N_DEV - 1)

        def make_send(o):
            j = lax.rem(my + o, N_DEV)
            return pltpu.make_async_remote_copy(
                src_ref=x_ref.at[pl.ds(j * m_per, m_per), :],
                dst_ref=comm_ref.at[my],
                send_sem=send_sems.at[o - 1],
                recv_sem=recv_sems.at[my],
                device_id=(j,),
                device_id_type=pl.DeviceIdType.MESH,
            )

        send_r = make_send(1)
        send_l = make_send(3)
        send_r.start()
        send_l.start()

        xl_copy = pltpu.make_async_copy(
            x_ref.at[pl.ds(my * m_per, m_per), :], xl_ref, xsem)
        xl_copy.start()
        w_copies = []
        for d in range(N_DEV):
            c = pltpu.make_async_copy(
                w_ref.at[pl.ds(d * k_shard, k_shard), :],
                wv_ref.at[d], wsems.at[d])
            c.start()
            w_copies.append(c)

        def chunk_dot(a, b):
            return lax.dot_general(
                a, b, (((1,), (0,)), ((), ())),
                preferred_element_type=jnp.int32,
            )

        xl_copy.wait()
        for c in w_copies:
            c.wait()
        acc_ref[...] = chunk_dot(xl_ref[...], wv_ref[my])

        send_r.wait_send()
        send_l.wait_send()
        send_d = make_send(2)
        send_d.start()

        def recv_from(d):
            return pltpu.make_async_remote_copy(
                src_ref=comm_ref.at[d],
                dst_ref=comm_ref.at[d],
                send_sem=send_sems.at[0],
                recv_sem=recv_sems.at[d],
                device_id=(d,),
                device_id_type=pl.DeviceIdType.MESH,
            )

        for o in (1, 3):
            d = lax.rem(my + N_DEV - o, N_DEV)
            recv_from(d).wait_recv()
            acc_ref[...] += chunk_dot(comm_ref[d], wv_ref[d])

        d2 = lax.rem(my + 2, N_DEV)
        recv_from(d2).wait_recv()
        scale = sx_ref[0] * sw_ref[0]
        out_dmas = []
        for c in range(N_COL_BLOCKS):
            cs = pl.ds(c * n_blk, n_blk)
            acc_ref[:, cs] += chunk_dot(comm_ref[d2], wv_ref[d2, :, cs])
            y = acc_ref[:, cs].astype(jnp.float32) * scale
            epi_ref[:, cs] = y * (1.0 / (1.0 + jnp.exp(-jnp.clip(y, -60.0, 60.0))))
            dma = pltpu.make_async_copy(
                epi_ref.at[:, cs], out_ref.at[:, cs], osems.at[c])
            dma.start()
            out_dmas.append(dma)

        for dma in out_dmas:
            dma.wait()
        send_d.wait_send()

    return pl.pallas_call(
        body,
        out_shape=jax.ShapeDtypeStruct((m_per, n), jnp.float32),
        in_specs=[
            pl.BlockSpec(memory_space=pltpu.MemorySpace.HBM),
            pl.BlockSpec(memory_space=pltpu.MemorySpace.HBM),
            pl.BlockSpec(memory_space=pltpu.SMEM),
            pl.BlockSpec(memory_space=pltpu.SMEM),
        ],
        out_specs=pl.BlockSpec(memory_space=pltpu.MemorySpace.HBM),
        scratch_shapes=[
            pltpu.VMEM((N_DEV, m_per, k_shard), jnp.int8),
            pltpu.VMEM((N_DEV, k_shard, n), jnp.int8),
            pltpu.VMEM((m_per, k_shard), jnp.int8),
            pltpu.VMEM((m_per, n), jnp.int32),
            pltpu.VMEM((m_per, n), jnp.float32),
            pltpu.SemaphoreType.DMA((N_DEV - 1,)),
            pltpu.SemaphoreType.DMA((N_DEV,)),
            pltpu.SemaphoreType.DMA((N_DEV,)),
            pltpu.SemaphoreType.DMA,
            pltpu.SemaphoreType.DMA((N_COL_BLOCKS,)),
        ],
        compiler_params=pltpu.CompilerParams(collective_id=0),
    )(x, w_mat, scale_x, scale_w)


# device time: 44470 ns/iter; 1.0402x vs baseline; 1.0402x over previous
import jax
import jax.numpy as jnp
from jax import lax
from jax.experimental import pallas as pl
from jax.experimental.pallas import tpu as pltpu

N_DEV = 4


def kernel(x, w_mat, scale_x, scale_w):
    m_total, k_shard = x.shape
    k_total, n = w_mat.shape
    m_per = m_total // N_DEV
    m_h = m_per // 2
    n_h = n // 2

    def body(x_ref, w_ref, sx_ref, sw_ref, out_ref,
             comm_ref, wv_ref, xl_ref, acc_ref, epi_ref,
             send_sems, recv_sems, dsems, wsems, xsem, osems):
        my = lax.axis_index("i")

        barrier_sem = pltpu.get_barrier_semaphore()
        for o in range(1, N_DEV):
            peer = lax.rem(my + o, N_DEV)
            pl.semaphore_signal(
                barrier_sem, inc=1,
                device_id=(peer,), device_id_type=pl.DeviceIdType.MESH,
            )
        pl.semaphore_wait(barrier_sem, N_DEV - 1)

        def make_send(o, sem_i):
            j = lax.rem(my + o, N_DEV)
            return pltpu.make_async_remote_copy(
                src_ref=x_ref.at[pl.ds(j * m_per, m_per), :],
                dst_ref=comm_ref.at[my],
                send_sem=send_sems.at[sem_i],
                recv_sem=recv_sems.at[my],
                device_id=(j,),
                device_id_type=pl.DeviceIdType.MESH,
            )

        send_r = make_send(1, 0)
        send_l = make_send(3, 1)
        send_r.start()
        send_l.start()

        def make_diag_send(h):
            j = lax.rem(my + 2, N_DEV)
            return pltpu.make_async_remote_copy(
                src_ref=x_ref.at[pl.ds(j * m_per + h * m_h, m_h), :],
                dst_ref=comm_ref.at[my, pl.ds(h * m_h, m_h), :],
                send_sem=send_sems.at[2 + h],
                recv_sem=dsems.at[h],
                device_id=(j,),
                device_id_type=pl.DeviceIdType.MESH,
            )

        xl_copy = pltpu.make_async_copy(
            x_ref.at[pl.ds(my * m_per, m_per), :], xl_ref, xsem)
        xl_copy.start()
        w_copies = []
        for d in range(N_DEV):
            c = pltpu.make_async_copy(
                w_ref.at[pl.ds(d * k_shard, k_shard), :],
                wv_ref.at[d], wsems.at[d])
            c.start()
            w_copies.append(c)

        def chunk_dot(a, b):
            return lax.dot_general(
                a, b, (((1,), (0,)), ((), ())),
                preferred_element_type=jnp.int32,
            )

        xl_copy.wait()
        for c in w_copies:
            c.wait()
        acc_ref[...] = chunk_dot(xl_ref[...], wv_ref[my])

        send_r.wait_send()
        send_l.wait_send()
        diag0 = make_diag_send(0)
        diag0.start()

        def recv_from(d):
            return pltpu.make_async_remote_copy(
                src_ref=comm_ref.at[d],
                dst_ref=comm_ref.at[d],
                send_sem=send_sems.at[0],
                recv_sem=recv_sems.at[d],
                device_id=(d,),
                device_id_type=pl.DeviceIdType.MESH,
            )

        def recv_diag(h):
            d2 = lax.rem(my + 2, N_DEV)
            return pltpu.make_async_remote_copy(
                src_ref=comm_ref.at[d2, pl.ds(h * m_h, m_h), :],
                dst_ref=comm_ref.at[d2, pl.ds(h * m_h, m_h), :],
                send_sem=send_sems.at[0],
                recv_sem=dsems.at[h],
                device_id=(d2,),
                device_id_type=pl.DeviceIdType.MESH,
            )

        d_left = lax.rem(my + N_DEV - 1, N_DEV)
        d_right = lax.rem(my + 1, N_DEV)
        recv_from(d_left).wait_recv()
        acc_ref[...] += chunk_dot(comm_ref[d_left], wv_ref[d_left])

        diag0.wait_send()
        diag1 = make_diag_send(1)
        diag1.start()

        recv_from(d_right).wait_recv()
        acc_ref[...] += chunk_dot(comm_ref[d_right], wv_ref[d_right])

        d2 = lax.rem(my + 2, N_DEV)
        scale = sx_ref[0] * sw_ref[0]
        out_dmas = []

        def epi_block(rs, cs, sem_i):
            y = acc_ref[rs, cs].astype(jnp.float32) * scale
            epi_ref[rs, cs] = y * (1.0 / (1.0 + jnp.exp(-jnp.clip(y, -60.0, 60.0))))
            dma = pltpu.make_async_copy(
                epi_ref.at[rs, cs], out_ref.at[rs, cs], osems.at[sem_i])
            dma.start()
            out_dmas.append(dma)

        for h in range(2):
            rs = pl.ds(h * m_h, m_h)
            recv_diag(h).wait_recv()
            acc_ref[rs, :] += chunk_dot(comm_ref[d2, rs, :], wv_ref[d2])
            epi_block(rs, pl.ds(0, n_h), 2 * h)
            epi_block(rs, pl.ds(n_h, n_h), 2 * h + 1)

        for dma in out_dmas:
            dma.wait()
        diag1.wait_send()

    return pl.pallas_call(
        body,
        out_shape=jax.ShapeDtypeStruct((m_per, n), jnp.float32),
        in_specs=[
            pl.BlockSpec(memory_space=pltpu.MemorySpace.HBM),
            pl.BlockSpec(memory_space=pltpu.MemorySpace.HBM),
            pl.BlockSpec(memory_space=pltpu.SMEM),
            pl.BlockSpec(memory_space=pltpu.SMEM),
        ],
        out_specs=pl.BlockSpec(memory_space=pltpu.MemorySpace.HBM),
        scratch_shapes=[
            pltpu.VMEM((N_DEV, m_per, k_shard), jnp.int8),
            pltpu.VMEM((N_DEV, k_shard, n), jnp.int8),
            pltpu.VMEM((m_per, k_shard), jnp.int8),
            pltpu.VMEM((m_per, n), jnp.int32),
            pltpu.VMEM((m_per, n), jnp.float32),
            pltpu.SemaphoreType.DMA((4,)),
            pltpu.SemaphoreType.DMA((N_DEV,)),
            pltpu.SemaphoreType.DMA((2,)),
            pltpu.SemaphoreType.DMA((N_DEV,)),
            pltpu.SemaphoreType.DMA,
            pltpu.SemaphoreType.DMA((4,)),
        ],
        compiler_params=pltpu.CompilerParams(collective_id=0),
    )(x, w_mat, scale_x, scale_w)


# device time: 42937 ns/iter; 1.0773x vs baseline; 1.0357x over previous
import jax
import jax.numpy as jnp
from jax import lax
from jax.experimental import pallas as pl
from jax.experimental.pallas import tpu as pltpu

N_DEV = 4


def kernel(x, w_mat, scale_x, scale_w):
    m_total, k_shard = x.shape
    k_total, n = w_mat.shape
    m_per = m_total // N_DEV
    m_h = m_per // 2
    n_h = n // 2

    def body(x_ref, w_ref, sx_ref, sw_ref, out_ref,
             comm_ref, acc_ref, send_sems, recv_sems, dsems):
        my = lax.axis_index("i")

        barrier_sem = pltpu.get_barrier_semaphore()
        for o in range(1, N_DEV):
            peer = lax.rem(my + o, N_DEV)
            pl.semaphore_signal(
                barrier_sem, inc=1,
                device_id=(peer,), device_id_type=pl.DeviceIdType.MESH,
            )
        pl.semaphore_wait(barrier_sem, N_DEV - 1)

        def make_send(o, sem_i):
            j = lax.rem(my + o, N_DEV)
            return pltpu.make_async_remote_copy(
                src_ref=x_ref.at[pl.ds(j * m_per, m_per), :],
                dst_ref=comm_ref.at[my],
                send_sem=send_sems.at[sem_i],
                recv_sem=recv_sems.at[my],
                device_id=(j,),
                device_id_type=pl.DeviceIdType.MESH,
            )

        def make_diag_send(h):
            j = lax.rem(my + 2, N_DEV)
            return pltpu.make_async_remote_copy(
                src_ref=x_ref.at[pl.ds(j * m_per + h * m_h, m_h), :],
                dst_ref=comm_ref.at[my, pl.ds(h * m_h, m_h), :],
                send_sem=send_sems.at[2 + h],
                recv_sem=dsems.at[h],
                device_id=(j,),
                device_id_type=pl.DeviceIdType.MESH,
            )

        send_r = make_send(1, 0)
        send_l = make_send(3, 1)
        send_r.start()
        send_l.start()

        def chunk_dot(a, b):
            return lax.dot_general(
                a, b, (((1,), (0,)), ((), ())),
                preferred_element_type=jnp.int32,
            )

        def wslice(d):
            return w_ref[pl.ds(d * k_shard, k_shard), :]

        acc_ref[...] = chunk_dot(x_ref[pl.ds(my * m_per, m_per), :], wslice(my))

        send_r.wait_send()
        send_l.wait_send()
        diag0 = make_diag_send(0)
        diag0.start()

        def recv_from(d):
            return pltpu.make_async_remote_copy(
                src_ref=comm_ref.at[d],
                dst_ref=comm_ref.at[d],
                send_sem=send_sems.at[0],
                recv_sem=recv_sems.at[d],
                device_id=(d,),
                device_id_type=pl.DeviceIdType.MESH,
            )

        def recv_diag(h):
            d2 = lax.rem(my + 2, N_DEV)
            return pltpu.make_async_remote_copy(
                src_ref=comm_ref.at[d2, pl.ds(h * m_h, m_h), :],
                dst_ref=comm_ref.at[d2, pl.ds(h * m_h, m_h), :],
                send_sem=send_sems.at[0],
                recv_sem=dsems.at[h],
                device_id=(d2,),
                device_id_type=pl.DeviceIdType.MESH,
            )

        d_left = lax.rem(my + N_DEV - 1, N_DEV)
        d_right = lax.rem(my + 1, N_DEV)
        recv_from(d_left).wait_recv()
        acc_ref[...] += chunk_dot(comm_ref[d_left], wslice(d_left))

        diag0.wait_send()
        diag1 = make_diag_send(1)
        diag1.start()

        recv_from(d_right).wait_recv()
        acc_ref[...] += chunk_dot(comm_ref[d_right], wslice(d_right))

        d2 = lax.rem(my + 2, N_DEV)
        r0 = pl.ds(0, m_h)
        r1 = pl.ds(m_h, m_h)
        c0 = pl.ds(0, n_h)
        c1 = pl.ds(n_h, n_h)
        scale = sx_ref[0] * sw_ref[0]

        def silu_store(rs, cs):
            y = acc_ref[rs, cs].astype(jnp.float32) * scale
            out_ref[rs, cs] = y * (1.0 / (1.0 + jnp.exp(-jnp.clip(y, -60.0, 60.0))))

        recv_diag(0).wait_recv()
        acc_ref[r0, :] += chunk_dot(comm_ref[d2, r0, :], wslice(d2))
        recv_diag(1).wait_recv()
        acc_ref[r1, c0] += chunk_dot(
            comm_ref[d2, r1, :], w_ref[pl.ds(d2 * k_shard, k_shard), c0])
        silu_store(r0, c0)
        acc_ref[r1, c1] += chunk_dot(
            comm_ref[d2, r1, :], w_ref[pl.ds(d2 * k_shard, k_shard), c1])
        silu_store(r0, c1)
        silu_store(r1, c0)
        silu_store(r1, c1)

        diag1.wait_send()

    return pl.pallas_call(
        body,
        out_shape=jax.ShapeDtypeStruct((m_per, n), jnp.float32),
        in_specs=[
            pl.BlockSpec(memory_space=pltpu.VMEM),
            pl.BlockSpec(memory_space=pltpu.VMEM),
            pl.BlockSpec(memory_space=pltpu.SMEM),
            pl.BlockSpec(memory_space=pltpu.SMEM),
        ],
        out_specs=pl.BlockSpec(memory_space=pltpu.VMEM),
        scratch_shapes=[
            pltpu.VMEM((N_DEV, m_per, k_shard), jnp.int8),
            pltpu.VMEM((m_per, n), jnp.int32),
            pltpu.SemaphoreType.DMA((4,)),
            pltpu.SemaphoreType.DMA((N_DEV,)),
            pltpu.SemaphoreType.DMA((2,)),
        ],
        compiler_params=pltpu.CompilerParams(collective_id=0),
    )(x, w_mat, scale_x, scale_w)


# device time: 42386 ns/iter; 1.0913x vs baseline; 1.0130x over previous
import jax
import jax.numpy as jnp
from jax import lax
from jax.experimental import pallas as pl
from jax.experimental.pallas import tpu as pltpu

N_DEV = 4


def kernel(x, w_mat, scale_x, scale_w):
    m_total, k_shard = x.shape
    k_total, n = w_mat.shape
    m_per = m_total // N_DEV
    m_h = m_per // 2
    n_h = n // 2

    def body(x_ref, w_ref, sx_ref, sw_ref, out_ref,
             comm_ref, acc_ref, send_sems, recv_sems, dsems):
        my = lax.axis_index("i")

        barrier_sem = pltpu.get_barrier_semaphore()
        for o in range(1, N_DEV):
            peer = lax.rem(my + o, N_DEV)
            pl.semaphore_signal(
                barrier_sem, inc=1,
                device_id=(peer,), device_id_type=pl.DeviceIdType.MESH,
            )
        pl.semaphore_wait(barrier_sem, N_DEV - 1)

        def make_send(o, sem_i):
            j = lax.rem(my + o, N_DEV)
            return pltpu.make_async_remote_copy(
                src_ref=x_ref.at[pl.ds(j * m_per, m_per), :],
                dst_ref=comm_ref.at[my],
                send_sem=send_sems.at[sem_i],
                recv_sem=recv_sems.at[my],
                device_id=(j,),
                device_id_type=pl.DeviceIdType.MESH,
            )

        def make_diag_send(h):
            j = lax.rem(my + 2, N_DEV)
            return pltpu.make_async_remote_copy(
                src_ref=x_ref.at[pl.ds(j * m_per + h * m_h, m_h), :],
                dst_ref=comm_ref.at[my, pl.ds(h * m_h, m_h), :],
                send_sem=send_sems.at[2 + h],
                recv_sem=dsems.at[h],
                device_id=(j,),
                device_id_type=pl.DeviceIdType.MESH,
            )

        send_r = make_send(1, 0)
        send_l = make_send(3, 1)
        send_r.start()
        send_l.start()

        def chunk_dot(a, b):
            return lax.dot_general(
                a, b, (((1,), (0,)), ((), ())),
                preferred_element_type=jnp.int32,
            )

        def wslice(d):
            return w_ref[pl.ds(d * k_shard, k_shard), :]

        acc_ref[...] = chunk_dot(x_ref[pl.ds(my * m_per, m_per), :], wslice(my))

        send_r.wait_send()
        send_l.wait_send()
        diag0 = make_diag_send(0)
        diag0.start()

        def recv_from(d):
            return pltpu.make_async_remote_copy(
                src_ref=comm_ref.at[d],
                dst_ref=comm_ref.at[d],
                send_sem=send_sems.at[0],
                recv_sem=recv_sems.at[d],
                device_id=(d,),
                device_id_type=pl.DeviceIdType.MESH,
            )

        def recv_diag(h):
            d2 = lax.rem(my + 2, N_DEV)
            return pltpu.make_async_remote_copy(
                src_ref=comm_ref.at[d2, pl.ds(h * m_h, m_h), :],
                dst_ref=comm_ref.at[d2, pl.ds(h * m_h, m_h), :],
                send_sem=send_sems.at[0],
                recv_sem=dsems.at[h],
                device_id=(d2,),
                device_id_type=pl.DeviceIdType.MESH,
            )

        d_left = lax.rem(my + N_DEV - 1, N_DEV)
        d_right = lax.rem(my + 1, N_DEV)
        recv_from(d_left).wait_recv()
        acc_ref[...] += chunk_dot(comm_ref[d_left], wslice(d_left))

        diag0.wait_send()
        diag1 = make_diag_send(1)
        diag1.start()

        recv_from(d_right).wait_recv()
        acc_ref[...] += chunk_dot(comm_ref[d_right], wslice(d_right))

        d2 = lax.rem(my + 2, N_DEV)
        r0 = pl.ds(0, m_h)
        r1 = pl.ds(m_h, m_h)
        c0 = pl.ds(0, n_h)
        c1 = pl.ds(n_h, n_h)
        scale = sx_ref[0] * sw_ref[0]

        def silu_store(rs, cs):
            y = acc_ref[rs, cs].astype(jnp.float32) * scale
            out_ref[rs, cs] = y * (0.5 * jnp.tanh(0.5 * y) + 0.5)

        recv_diag(0).wait_recv()
        acc_ref[r0, :] += chunk_dot(comm_ref[d2, r0, :], wslice(d2))
        recv_diag(1).wait_recv()
        acc_ref[r1, c0] += chunk_dot(
            comm_ref[d2, r1, :], w_ref[pl.ds(d2 * k_shard, k_shard), c0])
        silu_store(r0, c0)
        acc_ref[r1, c1] += chunk_dot(
            comm_ref[d2, r1, :], w_ref[pl.ds(d2 * k_shard, k_shard), c1])
        silu_store(r0, c1)
        silu_store(r1, c0)
        silu_store(r1, c1)

        diag1.wait_send()

    return pl.pallas_call(
        body,
        out_shape=jax.ShapeDtypeStruct((m_per, n), jnp.float32),
        in_specs=[
            pl.BlockSpec(memory_space=pltpu.VMEM),
            pl.BlockSpec(memory_space=pltpu.VMEM),
            pl.BlockSpec(memory_space=pltpu.SMEM),
            pl.BlockSpec(memory_space=pltpu.SMEM),
        ],
        out_specs=pl.BlockSpec(memory_space=pltpu.VMEM),
        scratch_shapes=[
            pltpu.VMEM((N_DEV, m_per, k_shard), jnp.int8),
            pltpu.VMEM((m_per, n), jnp.int32),
            pltpu.SemaphoreType.DMA((4,)),
            pltpu.SemaphoreType.DMA((N_DEV,)),
            pltpu.SemaphoreType.DMA((2,)),
        ],
        compiler_params=pltpu.CompilerParams(collective_id=0),
    )(x, w_mat, scale_x, scale_w)


# device time: 41717 ns/iter; 1.1088x vs baseline; 1.0160x over previous
import jax
import jax.numpy as jnp
from jax import lax
from jax.experimental import pallas as pl
from jax.experimental.pallas import tpu as pltpu

N_DEV = 4


def kernel(x, w_mat, scale_x, scale_w):
    m_total, k_shard = x.shape
    k_total, n = w_mat.shape
    m_per = m_total // N_DEV
    m_h = m_per // 2
    n_h = n // 2

    def body(x_ref, w_ref, sx_ref, sw_ref, out_ref,
             comm_ref, acc_ref, send_sems, recv_sems, dsems):
        my = lax.axis_index("i")

        barrier_sem = pltpu.get_barrier_semaphore()
        for o in range(1, N_DEV):
            peer = lax.rem(my + o, N_DEV)
            pl.semaphore_signal(
                barrier_sem, inc=1,
                device_id=(peer,), device_id_type=pl.DeviceIdType.MESH,
            )
        pl.semaphore_wait(barrier_sem, N_DEV - 1)

        def make_send(o, h):
            j = lax.rem(my + o, N_DEV)
            return pltpu.make_async_remote_copy(
                src_ref=x_ref.at[pl.ds(j * m_per + h * m_h, m_h), :],
                dst_ref=comm_ref.at[my, pl.ds(h * m_h, m_h), :],
                send_sem=send_sems.at[(o - 1) * 2 + h],
                recv_sem=recv_sems.at[my, h] if o != 2 else dsems.at[h],
                device_id=(j,),
                device_id_type=pl.DeviceIdType.MESH,
            )

        def recv_direct(d, h):
            return pltpu.make_async_remote_copy(
                src_ref=comm_ref.at[d, pl.ds(h * m_h, m_h), :],
                dst_ref=comm_ref.at[d, pl.ds(h * m_h, m_h), :],
                send_sem=send_sems.at[0],
                recv_sem=recv_sems.at[d, h],
                device_id=(d,),
                device_id_type=pl.DeviceIdType.MESH,
            )

        def recv_diag(h):
            d2 = lax.rem(my + 2, N_DEV)
            return pltpu.make_async_remote_copy(
                src_ref=comm_ref.at[d2, pl.ds(h * m_h, m_h), :],
                dst_ref=comm_ref.at[d2, pl.ds(h * m_h, m_h), :],
                send_sem=send_sems.at[0],
                recv_sem=dsems.at[h],
                device_id=(d2,),
                device_id_type=pl.DeviceIdType.MESH,
            )

        send_r0 = make_send(1, 0)
        send_l0 = make_send(3, 0)
        send_r0.start()
        send_l0.start()

        def chunk_dot(a, b):
            return lax.dot_general(
                a, b, (((1,), (0,)), ((), ())),
                preferred_element_type=jnp.int32,
            )

        def wslice(d):
            return w_ref[pl.ds(d * k_shard, k_shard), :]

        acc_ref[...] = chunk_dot(x_ref[pl.ds(my * m_per, m_per), :], wslice(my))

        send_r0.wait_send()
        send_r1 = make_send(1, 1)
        send_r1.start()
        send_l0.wait_send()
        send_l1 = make_send(3, 1)
        send_l1.start()

        d_left = lax.rem(my + N_DEV - 1, N_DEV)
        d_right = lax.rem(my + 1, N_DEV)
        r0 = pl.ds(0, m_h)
        r1 = pl.ds(m_h, m_h)

        recv_direct(d_left, 0).wait_recv()
        acc_ref[r0, :] += chunk_dot(comm_ref[d_left, r0, :], wslice(d_left))
        recv_direct(d_right, 0).wait_recv()
        acc_ref[r0, :] += chunk_dot(comm_ref[d_right, r0, :], wslice(d_right))

        send_r1.wait_send()
        send_l1.wait_send()
        diag0 = make_send(2, 0)
        diag0.start()

        recv_direct(d_left, 1).wait_recv()
        acc_ref[r1, :] += chunk_dot(comm_ref[d_left, r1, :], wslice(d_left))
        recv_direct(d_right, 1).wait_recv()
        acc_ref[r1, :] += chunk_dot(comm_ref[d_right, r1, :], wslice(d_right))

        diag0.wait_send()
        diag1 = make_send(2, 1)
        diag1.start()

        d2 = lax.rem(my + 2, N_DEV)
        c0 = pl.ds(0, n_h)
        c1 = pl.ds(n_h, n_h)
        scale = sx_ref[0] * sw_ref[0]

        def silu_store(rs, cs):
            y = acc_ref[rs, cs].astype(jnp.float32) * scale
            out_ref[rs, cs] = y * (0.5 * jnp.tanh(0.5 * y) + 0.5)

        recv_diag(0).wait_recv()
        acc_ref[r0, :] += chunk_dot(comm_ref[d2, r0, :], wslice(d2))
        silu_store(r0, c0)
        silu_store(r0, c1)

        recv_diag(1).wait_recv()
        acc_ref[r1, c0] += chunk_dot(
            comm_ref[d2, r1, :], w_ref[pl.ds(d2 * k_shard, k_shard), c0])
        silu_store(r1, c0)
        acc_ref[r1, c1] += chunk_dot(
            comm_ref[d2, r1, :], w_ref[pl.ds(d2 * k_shard, k_shard), c1])
        silu_store(r1, c1)

        diag1.wait_send()

    return pl.pallas_call(
        body,
        out_shape=jax.ShapeDtypeStruct((m_per, n), jnp.float32),
        in_specs=[
            pl.BlockSpec(memory_space=pltpu.VMEM),
            pl.BlockSpec(memory_space=pltpu.VMEM),
            pl.BlockSpec(memory_space=pltpu.SMEM),
            pl.BlockSpec(memory_space=pltpu.SMEM),
        ],
        out_specs=pl.BlockSpec(memory_space=pltpu.VMEM),
        scratch_shapes=[
            pltpu.VMEM((N_DEV, m_per, k_shard), jnp.int8),
            pltpu.VMEM((m_per, n), jnp.int32),
            pltpu.SemaphoreType.DMA((6,)),
            pltpu.SemaphoreType.DMA((N_DEV, 2)),
            pltpu.SemaphoreType.DMA((2,)),
        ],
        compiler_params=pltpu.CompilerParams(collective_id=0),
    )(x, w_mat, scale_x, scale_w)
